# Initial kernel scaffold; baseline (speedup 1.0000x reference)
import jax
import jax.numpy as jnp
from jax.experimental import pallas as pl


def kernel(node_repr, edges, W, b):
    raise NotImplementedError("write your pallas kernel here")


# speedup vs baseline: 23.6090x; 23.6090x over previous
import functools

import jax
import jax.numpy as jnp
from jax import lax
from jax.experimental import pallas as pl
from jax.experimental.pallas import tpu as pltpu
from jax.experimental.pallas import tpu_sc as plsc

B, S, D_IN, D_OUT, L, E = 8, 2048, 128, 128, 4, 32768

NC, NS = 2, 16
NW = NC * NS
CHUNK = 128
EPT = E // NW
ROWS_PT = EPT // CHUNK
NROW = S // NS


def _mm_body(x_ref, w_ref, bias_ref, h_ref):
    x = x_ref[0]
    w = w_ref[0]
    h = lax.dot_general(x, w, (((1,), (1,)), ((), ())),
                        preferred_element_type=jnp.float32)
    h_ref[0, 0] = h + bias_ref[pl.program_id(1)][None, :]


def _labelwise_transform(x, W, bias):
    return pl.pallas_call(
        _mm_body,
        grid=(B, L),
        in_specs=[
            pl.BlockSpec((1, S, D_IN), lambda b, l: (b, 0, 0)),
            pl.BlockSpec((1, D_OUT, D_IN), lambda b, l: (l, 0, 0)),
            pl.BlockSpec((L, D_OUT), lambda b, l: (0, 0)),
        ],
        out_specs=pl.BlockSpec((1, 1, S, D_OUT), lambda b, l: (b, l, 0, 0)),
        out_shape=jax.ShapeDtypeStruct((B, L, S, D_OUT), jnp.float32),
    )(x, W, bias)


def _sc_body(h_ref, src_ref, tgt_ref, lbl_ref, zero_ref, part_ref,
             sv, lv, tv, gidx, rows, zbuf, acc, sem):
    cid = lax.axis_index("c")
    sid = lax.axis_index("s")
    wid = cid * NS + sid
    row0 = wid * ROWS_PT
    acc_row0 = sid * NROW

    pltpu.sync_copy(zero_ref, zbuf)

    def batch_body(b, _):
        pltpu.sync_copy(zbuf, acc.at[pl.ds(acc_row0, NROW)])
        plsc.subcore_barrier()

        pltpu.sync_copy(src_ref.at[b, pl.ds(row0, ROWS_PT)], sv)
        pltpu.sync_copy(tgt_ref.at[b, pl.ds(row0, ROWS_PT)], tv)
        pltpu.sync_copy(lbl_ref.at[b, pl.ds(row0, ROWS_PT)], lv)

        base = b * (L * S)
        for j in range(ROWS_PT):
            for i in range(CHUNK // 16):
                s16 = sv[j, pl.ds(i * 16, 16)]
                l16 = lv[j, pl.ds(i * 16, 16)]
                gidx[j, pl.ds(i * 16, 16)] = l16 * S + s16 + base
            pltpu.async_copy(h_ref.at[gidx.at[j]], rows, sem).wait()
            pltpu.sync_copy(rows, acc.at[tv.at[j]], add=True)

        plsc.subcore_barrier()
        pltpu.sync_copy(acc.at[pl.ds(acc_row0, NROW)],
                        part_ref.at[cid, b, pl.ds(acc_row0, NROW)])
        plsc.subcore_barrier()
        return 0

    lax.fori_loop(0, B, batch_body, 0)


def _sc_scatter(h_flat, srcr, tgtr, lblr, zrow):
    mesh = plsc.VectorSubcoreMesh(core_axis_name="c", subcore_axis_name="s")
    k = pl.kernel(
        _sc_body,
        out_type=jax.ShapeDtypeStruct((NC, B, S, D_OUT), jnp.float32),
        mesh=mesh,
        scratch_types=[
            pltpu.VMEM((ROWS_PT, CHUNK), jnp.int32),
            pltpu.VMEM((ROWS_PT, CHUNK), jnp.int32),
            pltpu.VMEM((ROWS_PT, CHUNK), jnp.int32),
            pltpu.VMEM((ROWS_PT, CHUNK), jnp.int32),
            pltpu.VMEM((CHUNK, D_OUT), jnp.float32),
            pltpu.VMEM((NROW, D_OUT), jnp.float32),
            pltpu.VMEM_SHARED((S, D_OUT), jnp.float32),
            pltpu.SemaphoreType.DMA,
        ],
    )
    return k(h_flat, srcr, tgtr, lblr, zrow)


def _combine_body(p_ref, o_ref):
    o_ref[0] = jnp.maximum(p_ref[0, 0] + p_ref[1, 0], 0.0)


def _combine(part):
    return pl.pallas_call(
        _combine_body,
        grid=(B,),
        in_specs=[
            pl.BlockSpec((2, 1, S, D_OUT), lambda b: (0, b, 0, 0)),
        ],
        out_specs=pl.BlockSpec((1, S, D_OUT), lambda b: (b, 0, 0)),
        out_shape=jax.ShapeDtypeStruct((B, S, D_OUT), jnp.float32),
    )(part)


@jax.jit
def kernel(node_repr, edges, W, b):
    src = edges[..., 0].reshape(B, E // CHUNK, CHUNK)
    tgt = edges[..., 1].reshape(B, E // CHUNK, CHUNK)
    lbl = edges[..., 2].reshape(B, E // CHUNK, CHUNK)

    h = _labelwise_transform(node_repr, W, b)
    h_flat = h.reshape(B * L * S, D_OUT)

    zrow = jnp.zeros((NROW, D_OUT), dtype=jnp.float32)
    part = _sc_scatter(h_flat, src, tgt, lbl, zrow)
    return _combine(part)


# speedup vs baseline: 35.5496x; 1.5058x over previous
import jax
import jax.numpy as jnp
from jax import lax
from jax.experimental import pallas as pl
from jax.experimental.pallas import tpu as pltpu
from jax.experimental.pallas import tpu_sc as plsc

B, S, D_IN, D_OUT, L, E = 8, 2048, 128, 128, 4, 32768

NC, NS = 2, 16
BPC = B // NC
CHUNK = 128
EPT = E // NS
ROWS_PT = EPT // CHUNK
NROW = S // NS


def _mm_body(x_ref, w_ref, bias_ref, h_ref):
    x = x_ref[0]
    w = w_ref[0]
    h = lax.dot_general(x, w, (((1,), (1,)), ((), ())),
                        preferred_element_type=jnp.float32)
    h_ref[0, 0] = h + bias_ref[pl.program_id(1)][None, :]


def _labelwise_transform(x, W, bias):
    return pl.pallas_call(
        _mm_body,
        grid=(B, L),
        in_specs=[
            pl.BlockSpec((1, S, D_IN), lambda b, l: (b, 0, 0)),
            pl.BlockSpec((1, D_OUT, D_IN), lambda b, l: (l, 0, 0)),
            pl.BlockSpec((L, D_OUT), lambda b, l: (0, 0)),
        ],
        out_specs=pl.BlockSpec((1, 1, S, D_OUT), lambda b, l: (b, l, 0, 0)),
        out_shape=jax.ShapeDtypeStruct((B, L, S, D_OUT), jnp.float32),
    )(x, W, bias)


def _sc_body(h_ref, src_ref, tgt_ref, lbl_ref, zero_ref, out_ref,
             sv, lv, tv, gidx, rows0, rows1, zbuf, acc, sem0, sem1):
    cid = lax.axis_index("c")
    sid = lax.axis_index("s")
    row0 = sid * ROWS_PT
    acc_row0 = sid * NROW
    rows = (rows0, rows1)
    sems = (sem0, sem1)

    pltpu.sync_copy(zero_ref, zbuf)

    def batch_body(bi, _):
        b = cid * BPC + bi
        pltpu.sync_copy(zbuf, acc.at[pl.ds(acc_row0, NROW)])

        pltpu.sync_copy(src_ref.at[b, pl.ds(row0, ROWS_PT)], sv)
        pltpu.sync_copy(tgt_ref.at[b, pl.ds(row0, ROWS_PT)], tv)
        pltpu.sync_copy(lbl_ref.at[b, pl.ds(row0, ROWS_PT)], lv)

        base = b * (L * S)
        for j in range(ROWS_PT):
            for i in range(CHUNK // 16):
                s16 = sv[j, pl.ds(i * 16, 16)]
                l16 = lv[j, pl.ds(i * 16, 16)]
                gidx[j, pl.ds(i * 16, 16)] = l16 * S + s16 + base

        plsc.subcore_barrier()

        desc = pltpu.async_copy(h_ref.at[gidx.at[0]], rows[0], sems[0])
        for j in range(ROWS_PT):
            if j + 1 < ROWS_PT:
                nxt = pltpu.async_copy(
                    h_ref.at[gidx.at[j + 1]], rows[(j + 1) % 2],
                    sems[(j + 1) % 2])
            desc.wait()
            pltpu.sync_copy(rows[j % 2], acc.at[tv.at[j]], add=True)
            if j + 1 < ROWS_PT:
                desc = nxt

        plsc.subcore_barrier()

        pltpu.sync_copy(acc.at[pl.ds(acc_row0, NROW)], rows0)

        def relu_row(r, _):
            for i in range(D_OUT // 16):
                v = rows0[r, pl.ds(i * 16, 16)]
                rows0[r, pl.ds(i * 16, 16)] = jnp.maximum(v, 0.0)
            return 0

        lax.fori_loop(0, NROW, relu_row, 0)
        pltpu.sync_copy(rows0, out_ref.at[b, pl.ds(acc_row0, NROW)])
        plsc.subcore_barrier()
        return 0

    lax.fori_loop(0, BPC, batch_body, 0)


def _sc_scatter(h_flat, srcr, tgtr, lblr, zrow):
    mesh = plsc.VectorSubcoreMesh(core_axis_name="c", subcore_axis_name="s")
    k = pl.kernel(
        _sc_body,
        out_type=jax.ShapeDtypeStruct((B, S, D_OUT), jnp.float32),
        mesh=mesh,
        scratch_types=[
            pltpu.VMEM((ROWS_PT, CHUNK), jnp.int32),
            pltpu.VMEM((ROWS_PT, CHUNK), jnp.int32),
            pltpu.VMEM((ROWS_PT, CHUNK), jnp.int32),
            pltpu.VMEM((ROWS_PT, CHUNK), jnp.int32),
            pltpu.VMEM((CHUNK, D_OUT), jnp.float32),
            pltpu.VMEM((CHUNK, D_OUT), jnp.float32),
            pltpu.VMEM((NROW, D_OUT), jnp.float32),
            pltpu.VMEM_SHARED((S, D_OUT), jnp.float32),
            pltpu.SemaphoreType.DMA,
            pltpu.SemaphoreType.DMA,
        ],
    )
    return k(h_flat, srcr, tgtr, lblr, zrow)


@jax.jit
def kernel(node_repr, edges, W, b):
    src = edges[..., 0].reshape(B, E // CHUNK, CHUNK)
    tgt = edges[..., 1].reshape(B, E // CHUNK, CHUNK)
    lbl = edges[..., 2].reshape(B, E // CHUNK, CHUNK)

    h = _labelwise_transform(node_repr, W, b)
    h_flat = h.reshape(B * L * S, D_OUT)

    zrow = jnp.zeros((NROW, D_OUT), dtype=jnp.float32)
    return _sc_scatter(h_flat, src, tgt, lbl, zrow)
